# baseline (speedup 1.0000x reference)
import functools

import jax
import jax.numpy as jnp
from jax import lax
from jax.experimental import pallas as pl
from jax.experimental.pallas import tpu as pltpu
from jax.experimental.pallas import tpu_sc as plsc

NUM_TYPE = 6
NUM_DIR = 3
NTAB = NUM_TYPE * NUM_DIR
D = 64
E_TOTAL = 800000

NC = 2
NS = 16
NW = NC * NS

CHUNK = 160
NCHUNK_TOT = E_TOTAL // CHUNK
GROUPS = CHUNK // 16


def _sc_body(
    ea_hbm, comb_hbm, out_hbm,
    ea0_v, ea1_v, comb_v, addr_v, rows0_v, rows1_v,
    in_sem0, in_sem1, out_sem0, out_sem1,
):
    wid = lax.axis_index("s") * NC + lax.axis_index("c")
    ea_bufs = (ea0_v, ea1_v)
    rows_bufs = (rows0_v, rows1_v)
    in_sems = (in_sem0, in_sem1)
    out_sems = (out_sem0, out_sem1)

    pltpu.sync_copy(comb_hbm, comb_v)

    iota = lax.iota(jnp.int32, 16)
    zeros = jnp.zeros((16,), jnp.int32)
    ones = jnp.ones((16,), jnp.int32)
    k_iters = (NCHUNK_TOT + NW - 1) // NW

    def start_in_copy(k, half):
        c = wid + k * NW

        @pl.when(c < NCHUNK_TOT)
        def _():
            ebase = pl.multiple_of(c * CHUNK, 8)
            pltpu.async_copy(
                ea_hbm.at[pl.ds(ebase, CHUNK)], ea_bufs[half], in_sems[half]
            )

    def run_chunk(k, kk, half):
        ea_v = ea_bufs[half]
        rows_v = rows_bufs[half]
        c = wid + k * NW
        ebase = pl.multiple_of(c * CHUNK, 8)

        pltpu.make_async_copy(
            ea_hbm.at[pl.ds(0, CHUNK)], ea_v, in_sems[half]
        ).wait()
        start_in_copy(k + 1, half ^ 1)

        @pl.when(kk >= 1)
        def _drain():
            pltpu.make_async_copy(
                out_hbm.at[pl.ds(0, CHUNK)], rows_v, out_sems[half]
            ).wait()

        for g in range(GROUPS):
            rows = iota + g * 16
            a = plsc.load_gather(ea_v, [rows, zeros])
            b = plsc.load_gather(ea_v, [rows, ones])
            t = jnp.clip(a * 3 + b, 0, NTAB - 1)
            addr_v[pl.ds(g * 16, 16)] = t * D

        def d_body(d, carry2):
            rot = (iota + d) & (D - 1)
            for blk in range(0, GROUPS, 8):
                gs = range(blk, min(blk + 8, GROUPS))
                vals = []
                for g in gs:
                    av = addr_v[pl.ds(g * 16, 16)]
                    vals.append(plsc.load_gather(comb_v, [av + rot]))
                for g, v in zip(gs, vals):
                    plsc.store_scatter(rows_v, [iota + g * 16, rot], v)
            return carry2

        lax.fori_loop(0, D, d_body, 0)

        pltpu.async_copy(rows_v, out_hbm.at[pl.ds(ebase, CHUNK)], out_sems[half])

    def pair_body(kk, carry):
        for half in (0, 1):
            k = kk * 2 + half
            c = wid + k * NW

            @pl.when(c < NCHUNK_TOT)
            def _():
                run_chunk(k, kk, half)

        return carry

    start_in_copy(0, 0)
    lax.fori_loop(0, (k_iters + 1) // 2, pair_body, 0)

    for half in (0, 1):
        pltpu.make_async_copy(
            out_hbm.at[pl.ds(0, CHUNK)], rows_bufs[half], out_sems[half]
        ).wait()


@jax.jit
def _encode(edge_attr_i32, comb):
    mesh = plsc.VectorSubcoreMesh(
        core_axis_name="c", subcore_axis_name="s", num_cores=NC, num_subcores=NS
    )
    fn = pl.kernel(
        _sc_body,
        out_type=jax.ShapeDtypeStruct((E_TOTAL, D), jnp.float32),
        mesh=mesh,
        compiler_params=pltpu.CompilerParams(
            needs_layout_passes=False, use_tc_tiling_on_sc=True
        ),
        scratch_types=[
            pltpu.VMEM((CHUNK, 2), jnp.int32),
            pltpu.VMEM((CHUNK, 2), jnp.int32),
            pltpu.VMEM((NTAB * D,), jnp.float32),
            pltpu.VMEM((CHUNK,), jnp.int32),
            pltpu.VMEM((CHUNK, D), jnp.float32),
            pltpu.VMEM((CHUNK, D), jnp.float32),
            pltpu.SemaphoreType.DMA,
            pltpu.SemaphoreType.DMA,
            pltpu.SemaphoreType.DMA,
            pltpu.SemaphoreType.DMA,
        ],
    )
    return fn(edge_attr_i32, comb)


def kernel(edge_attr, W):
    Wt = W.T.astype(jnp.float32)
    comb = (Wt[:NUM_TYPE, None, :] + Wt[None, NUM_TYPE:, :]).reshape(NTAB * D)
    return _encode(edge_attr.astype(jnp.int32), comb)


# speedup vs baseline: 1.4435x; 1.4435x over previous
import functools

import jax
import jax.numpy as jnp
from jax import lax
from jax.experimental import pallas as pl
from jax.experimental.pallas import tpu as pltpu
from jax.experimental.pallas import tpu_sc as plsc

NUM_TYPE = 6
NUM_DIR = 3
NTAB = NUM_TYPE * NUM_DIR
D = 64
DP = 128
E_TOTAL = 800000

NC = 2
NS = 16
NW = NC * NS

CHUNK = 400
NCHUNK_TOT = E_TOTAL // CHUNK
GROUPS = CHUNK // 16


def _sc_body(
    a_hbm, b_hbm, comb_hbm, out_hbm,
    a0_v, a1_v, b0_v, b1_v, comb_v, addr_v, rows0_v, rows1_v,
    in_sem0, in_sem1, sem0, sem1,
):
    wid = lax.axis_index("s") * NC + lax.axis_index("c")
    a_bufs = (a0_v, a1_v)
    b_bufs = (b0_v, b1_v)
    rows_bufs = (rows0_v, rows1_v)
    in_sems = (in_sem0, in_sem1)
    sems = (sem0, sem1)

    pltpu.sync_copy(comb_hbm, comb_v)

    iota = lax.iota(jnp.int32, 16)
    k_iters = (NCHUNK_TOT + NW - 1) // NW

    def start_in_copy(k, half):
        c = wid + k * NW

        @pl.when(c < NCHUNK_TOT)
        def _():
            ebase = pl.multiple_of(c * CHUNK, 8)
            pltpu.async_copy(a_hbm.at[pl.ds(ebase, CHUNK)], a_bufs[half], in_sems[half])
            pltpu.async_copy(b_hbm.at[pl.ds(ebase, CHUNK)], b_bufs[half], in_sems[half])

    def run_chunk(k, kk, half):
        a_v = a_bufs[half]
        b_v = b_bufs[half]
        rows_v = rows_bufs[half]
        sem = sems[half]
        c = wid + k * NW
        ebase = pl.multiple_of(c * CHUNK, 8)

        pltpu.make_async_copy(a_hbm.at[pl.ds(0, CHUNK)], a_v, in_sems[half]).wait()
        pltpu.make_async_copy(b_hbm.at[pl.ds(0, CHUNK)], b_v, in_sems[half]).wait()
        start_in_copy(k + 1, half ^ 1)

        @pl.when(kk >= 1)
        def _drain():
            pltpu.make_async_copy(
                out_hbm.at[pl.ds(0, CHUNK)], rows_v, sem
            ).wait()

        for g in range(GROUPS):
            a = a_v[pl.ds(g * 16, 16)]
            b = b_v[pl.ds(g * 16, 16)]
            t = jnp.clip(a * 3 + b, 0, NTAB - 1)
            addr_v[pl.ds(g * 16, 16)] = t * D

        def d_body(d, carry2):
            rot = (iota + d) & (D - 1)
            for blk in range(0, GROUPS, 8):
                gs = range(blk, min(blk + 8, GROUPS))
                vals = []
                for g in gs:
                    av = addr_v[pl.ds(g * 16, 16)]
                    vals.append(plsc.load_gather(comb_v, [av + rot]))
                for g, v in zip(gs, vals):
                    plsc.store_scatter(rows_v, [iota + g * 16, rot], v)
            return carry2

        lax.fori_loop(0, D, d_body, 0)

        pltpu.async_copy(rows_v, out_hbm.at[pl.ds(ebase, CHUNK)], sem)

    def pair_body(kk, carry):
        for half in (0, 1):
            k = kk * 2 + half
            c = wid + k * NW

            @pl.when(c < NCHUNK_TOT)
            def _():
                run_chunk(k, kk, half)

        return carry

    start_in_copy(0, 0)
    lax.fori_loop(0, (k_iters + 1) // 2, pair_body, 0)

    for half in (0, 1):
        pltpu.make_async_copy(
            out_hbm.at[pl.ds(0, CHUNK)], rows_bufs[half], sems[half]
        ).wait()


@jax.jit
def _encode(a_col, b_col, comb):
    mesh = plsc.VectorSubcoreMesh(
        core_axis_name="c", subcore_axis_name="s", num_cores=NC, num_subcores=NS
    )
    fn = pl.kernel(
        _sc_body,
        out_type=jax.ShapeDtypeStruct((E_TOTAL, D), jnp.float32),
        mesh=mesh,
        compiler_params=pltpu.CompilerParams(
            needs_layout_passes=False, use_tc_tiling_on_sc=True
        ),
        scratch_types=[
            pltpu.VMEM((CHUNK,), jnp.int32),
            pltpu.VMEM((CHUNK,), jnp.int32),
            pltpu.VMEM((CHUNK,), jnp.int32),
            pltpu.VMEM((CHUNK,), jnp.int32),
            pltpu.VMEM((NTAB * D,), jnp.float32),
            pltpu.VMEM((CHUNK,), jnp.int32),
            pltpu.VMEM((CHUNK, D), jnp.float32),
            pltpu.VMEM((CHUNK, D), jnp.float32),
            pltpu.SemaphoreType.DMA,
            pltpu.SemaphoreType.DMA,
            pltpu.SemaphoreType.DMA,
            pltpu.SemaphoreType.DMA,
        ],
    )
    return fn(a_col, b_col, comb)


def kernel(edge_attr, W):
    Wt = W.T.astype(jnp.float32)
    comb = (Wt[:NUM_TYPE, None, :] + Wt[None, NUM_TYPE:, :]).reshape(NTAB * D)
    ea = edge_attr.astype(jnp.int32)
    return _encode(ea[:, 0], ea[:, 1], comb)
